# Initial kernel scaffold; baseline (speedup 1.0000x reference)
import jax
import jax.numpy as jnp
from jax.experimental import pallas as pl


def kernel(x, last_update, edge_index, t, msg, W_time, b_time, Wq, bq, Wk, bk, Wv, bv, We, Wskip, bskip):
    raise NotImplementedError("write your pallas kernel here")


# speedup vs baseline: 5.0889x; 5.0889x over previous
import functools

import jax
import jax.numpy as jnp
from jax import lax
from jax.experimental import pallas as pl
from jax.experimental.pallas import tpu as pltpu
from jax.experimental.pallas import tpu_sc as plsc

N = 10000
E = 320000
IN = 128
OUT = 128
HEADS = 2
C = OUT // HEADS
MSG_DIM = 16
TIME_DIM = 100
KVW = 384

NC = 2
NS = 16
NW = NC * NS
PER_TILE = E // NW
PER_TILE_1C = E // NS
CHUNK = 80

BE = 2000
BN = 2000


def _tables_body(x_ref, w_ref, b_ref, lu_ref, q_ref, kv_ref, sk_ref):
    acc = jnp.dot(x_ref[...], w_ref[...], preferred_element_type=jnp.float32)
    acc = acc + b_ref[...]
    q_ref[...] = acc[:, 0:128]
    kv_ref[:, 0:256] = acc[:, 128:384]
    kv_ref[:, 256:257] = lu_ref[...]
    kv_ref[:, 257:KVW] = jnp.zeros((acc.shape[0], KVW - 257), jnp.float32)
    sk_ref[...] = acc[:, 384:512]


def _tables(x, wcat, bcat, lu2d):
    return pl.pallas_call(
        _tables_body,
        grid=(N // BN,),
        in_specs=[
            pl.BlockSpec((BN, IN), lambda i: (i, 0)),
            pl.BlockSpec((IN, 512), lambda i: (0, 0)),
            pl.BlockSpec((1, 512), lambda i: (0, 0)),
            pl.BlockSpec((BN, 1), lambda i: (i, 0)),
        ],
        out_specs=[
            pl.BlockSpec((BN, 128), lambda i: (i, 0)),
            pl.BlockSpec((BN, KVW), lambda i: (i, 0)),
            pl.BlockSpec((BN, 128), lambda i: (i, 0)),
        ],
        out_shape=[
            jax.ShapeDtypeStruct((N, 128), jnp.float32),
            jax.ShapeDtypeStruct((N, KVW), jnp.float32),
            jax.ShapeDtypeStruct((N, 128), jnp.float32),
        ],
    )(x, wcat, bcat, lu2d)


def _gather_body(qt, kvt, src, dst,
                 qd_out, kvs_out,
                 sidx, didx, qbuf, kvbuf, sem_q, sem_kv):
    wid = lax.axis_index("s") * NC + lax.axis_index("c")
    base = wid * PER_TILE

    def chunk(i, carry):
        off = base + i * CHUNK
        pltpu.sync_copy(src.at[pl.ds(off, CHUNK)], sidx)
        pltpu.sync_copy(dst.at[pl.ds(off, CHUNK)], didx)
        cp_q = pltpu.async_copy(qt.at[didx], qbuf, sem_q)
        cp_kv = pltpu.async_copy(kvt.at[sidx], kvbuf, sem_kv)
        cp_q.wait()
        cp_kv.wait()
        pltpu.sync_copy(qbuf, qd_out.at[pl.ds(off, CHUNK)])
        pltpu.sync_copy(kvbuf, kvs_out.at[pl.ds(off, CHUNK)])
        return carry

    lax.fori_loop(0, PER_TILE // CHUNK, chunk, 0)


def _gather(qt, kvt, src, dst):
    mesh = plsc.VectorSubcoreMesh(core_axis_name="c", subcore_axis_name="s")
    fn = functools.partial(
        pl.kernel,
        out_type=(
            jax.ShapeDtypeStruct((E, 128), jnp.float32),
            jax.ShapeDtypeStruct((E, KVW), jnp.float32),
        ),
        mesh=mesh,
        scratch_types=[
            pltpu.VMEM((CHUNK,), jnp.int32),
            pltpu.VMEM((CHUNK,), jnp.int32),
            pltpu.VMEM((CHUNK, 128), jnp.float32),
            pltpu.VMEM((CHUNK, KVW), jnp.float32),
            pltpu.SemaphoreType.DMA,
            pltpu.SemaphoreType.DMA,
        ],
    )(_gather_body)
    return fn(qt, kvt, src, dst)


def _edge_body(qd_ref, kvs_ref, t_ref, msg_ref,
               wtr_ref, btr_ref, wet_ref, wem_ref, shead_ref, bh_ref,
               pay0_ref, pay1_ref):
    kvs = kvs_ref[...]
    rel = kvs[:, 256:257] - t_ref[...]
    enc = jnp.cos(rel * wtr_ref[...] + btr_ref[...])
    e = jnp.dot(enc, wet_ref[...], preferred_element_type=jnp.float32)
    e = e + jnp.dot(msg_ref[...], wem_ref[...], preferred_element_type=jnp.float32)
    k = kvs[:, 0:128] + e
    v = kvs[:, 128:256] + e
    alpha = jnp.dot(qd_ref[...] * k, shead_ref[...],
                    preferred_element_type=jnp.float32) * 0.125
    ex = jnp.exp(alpha)
    exb = jnp.dot(ex, bh_ref[...], preferred_element_type=jnp.float32)
    pm = exb * v
    zer64 = jnp.zeros((pm.shape[0], 63), jnp.float32)
    pay0_ref[:, 0:64] = pm[:, 0:64]
    pay0_ref[:, 64:65] = ex[:, 0:1]
    pay0_ref[:, 65:128] = zer64
    pay1_ref[:, 0:64] = pm[:, 64:128]
    pay1_ref[:, 64:65] = ex[:, 1:2]
    pay1_ref[:, 65:128] = zer64


def _edges(qd, kvs, tf, msg, wtr, btr, wet, wem, shead, bh):
    return pl.pallas_call(
        _edge_body,
        grid=(E // BE,),
        in_specs=[
            pl.BlockSpec((BE, 128), lambda i: (i, 0)),
            pl.BlockSpec((BE, KVW), lambda i: (i, 0)),
            pl.BlockSpec((BE, 1), lambda i: (i, 0)),
            pl.BlockSpec((BE, MSG_DIM), lambda i: (i, 0)),
            pl.BlockSpec((1, 128), lambda i: (0, 0)),
            pl.BlockSpec((1, 128), lambda i: (0, 0)),
            pl.BlockSpec((128, 128), lambda i: (0, 0)),
            pl.BlockSpec((MSG_DIM, 128), lambda i: (0, 0)),
            pl.BlockSpec((128, HEADS), lambda i: (0, 0)),
            pl.BlockSpec((HEADS, 128), lambda i: (0, 0)),
        ],
        out_specs=[
            pl.BlockSpec((BE, 128), lambda i: (i, 0)),
            pl.BlockSpec((BE, 128), lambda i: (i, 0)),
        ],
        out_shape=[
            jax.ShapeDtypeStruct((E, 128), jnp.float32),
            jax.ShapeDtypeStruct((E, 128), jnp.float32),
        ],
    )(qd, kvs, tf, msg, wtr, btr, wet, wem, shead, bh)


def _scatter_body(pay0, pay1, dst, zer, out0, out1, table, idxv, pbuf):
    cid = lax.axis_index("c")
    sid = lax.axis_index("s")

    @pl.when(sid == 0)
    def _init():
        pltpu.sync_copy(zer, table)

    plsc.subcore_barrier()

    def chunk(i, carry):
        off = sid * PER_TILE_1C + i * CHUNK
        pltpu.sync_copy(dst.at[pl.ds(off, CHUNK)], idxv)

        @pl.when(cid == 0)
        def _l0():
            pltpu.sync_copy(pay0.at[pl.ds(off, CHUNK)], pbuf)

        @pl.when(cid == 1)
        def _l1():
            pltpu.sync_copy(pay1.at[pl.ds(off, CHUNK)], pbuf)

        pltpu.sync_copy(pbuf, table.at[idxv], add=True)
        return carry

    lax.fori_loop(0, PER_TILE_1C // CHUNK, chunk, 0)
    plsc.subcore_barrier()

    @pl.when((sid == 0) & (cid == 0))
    def _dump0():
        pltpu.sync_copy(table, out0)

    @pl.when((sid == 0) & (cid == 1))
    def _dump1():
        pltpu.sync_copy(table, out1)


def _scatter(pay0, pay1, dst, zer):
    mesh = plsc.VectorSubcoreMesh(core_axis_name="c", subcore_axis_name="s")
    fn = functools.partial(
        pl.kernel,
        out_type=(
            jax.ShapeDtypeStruct((N, 128), jnp.float32),
            jax.ShapeDtypeStruct((N, 128), jnp.float32),
        ),
        mesh=mesh,
        scratch_types=[
            pltpu.VMEM_SHARED((N, 128), jnp.float32),
            pltpu.VMEM((CHUNK,), jnp.int32),
            pltpu.VMEM((CHUNK, 128), jnp.float32),
        ],
    )(_scatter_body)
    return fn(pay0, pay1, dst, zer)


def _final_body(p0_ref, p1_ref, sk_ref, m0_ref, m1_ref, d0_ref, d1_ref, out_ref):
    p0 = p0_ref[...]
    p1 = p1_ref[...]
    main = jnp.dot(p0, m0_ref[...], preferred_element_type=jnp.float32)
    main = main + jnp.dot(p1, m1_ref[...], preferred_element_type=jnp.float32)
    den = jnp.dot(p0, d0_ref[...], preferred_element_type=jnp.float32)
    den = den + jnp.dot(p1, d1_ref[...], preferred_element_type=jnp.float32)
    agg = jnp.where(den > 0.0, main / den, 0.0)
    out_ref[...] = agg + sk_ref[...]


def _final(p0, p1, skip, m0, m1, d0, d1):
    return pl.pallas_call(
        _final_body,
        grid=(N // BN,),
        in_specs=[
            pl.BlockSpec((BN, 128), lambda i: (i, 0)),
            pl.BlockSpec((BN, 128), lambda i: (i, 0)),
            pl.BlockSpec((BN, 128), lambda i: (i, 0)),
            pl.BlockSpec((128, 128), lambda i: (0, 0)),
            pl.BlockSpec((128, 128), lambda i: (0, 0)),
            pl.BlockSpec((128, 128), lambda i: (0, 0)),
            pl.BlockSpec((128, 128), lambda i: (0, 0)),
        ],
        out_specs=pl.BlockSpec((BN, 128), lambda i: (i, 0)),
        out_shape=jax.ShapeDtypeStruct((N, 128), jnp.float32),
    )(p0, p1, skip, m0, m1, d0, d1)


def kernel(x, last_update, edge_index, t, msg, W_time, b_time,
           Wq, bq, Wk, bk, Wv, bv, We, Wskip, bskip):
    f32 = jnp.float32
    src = edge_index[0].astype(jnp.int32)
    dst = edge_index[1].astype(jnp.int32)
    lu2d = last_update.astype(f32)[:, None]
    tf = t.astype(f32)[:, None]

    wcat = jnp.concatenate([Wq, Wk, Wv, Wskip], axis=0).T.astype(f32)
    bcat = jnp.concatenate([bq, bk, bv, bskip])[None, :].astype(f32)

    wtr = jnp.zeros((1, 128), f32).at[0, :TIME_DIM].set(W_time[:, 0])
    btr = jnp.zeros((1, 128), f32).at[0, :TIME_DIM].set(b_time)
    wet = jnp.zeros((128, 128), f32).at[:TIME_DIM, :].set(We.T[:TIME_DIM, :])
    wem = We.T[TIME_DIM:, :].astype(f32)

    cols = jnp.arange(128) // C
    shead = (cols[:, None] == jnp.arange(HEADS)[None, :]).astype(f32)
    bh = shead.T

    i128 = jnp.arange(128)
    eye = jnp.eye(128, dtype=f32)
    m0 = jnp.where((i128[None, :] < 64) & (i128[:, None] == i128[None, :]), eye, 0.0)
    m1 = jnp.zeros((128, 128), f32).at[jnp.arange(64), jnp.arange(64) + 64].set(1.0)
    d0 = jnp.zeros((128, 128), f32).at[64, :].set((i128 < 64).astype(f32))
    d1 = jnp.zeros((128, 128), f32).at[64, :].set((i128 >= 64).astype(f32))

    zer = jnp.zeros((N, 128), f32)

    qt, kvt, skip = _tables(x, wcat, bcat, lu2d)
    qd, kvs = _gather(qt, kvt, src, dst)
    pay0, pay1 = _edges(qd, kvs, tf, msg, wtr, btr, wet, wem, shead, bh)
    p0, p1 = _scatter(pay0, pay1, dst, zer)
    return _final(p0, p1, skip, m0, m1, d0, d1)


# speedup vs baseline: 5.7590x; 1.1317x over previous
import functools

import jax
import jax.numpy as jnp
from jax import lax
from jax.experimental import pallas as pl
from jax.experimental.pallas import tpu as pltpu
from jax.experimental.pallas import tpu_sc as plsc

N = 10000
E = 320000
IN = 128
OUT = 128
HEADS = 2
C = OUT // HEADS
MSG_DIM = 16
TIME_DIM = 100
KVW = 384

NC = 2
NS = 16
NW = NC * NS
PER_TILE = E // NW
PER_TILE_1C = E // NS
CHUNK = 96
G_NCH = PER_TILE // CHUNK
G_TAIL = PER_TILE - G_NCH * CHUNK
S_NCH = PER_TILE_1C // CHUNK
S_TAIL = PER_TILE_1C - S_NCH * CHUNK

BE = 2000
BN = 2000


def _tables_body(x_ref, w_ref, b_ref, lu_ref, q_ref, kv_ref, sk_ref):
    acc = jnp.dot(x_ref[...], w_ref[...], preferred_element_type=jnp.float32)
    acc = acc + b_ref[...]
    q_ref[...] = acc[:, 0:128]
    kv_ref[:, 0:256] = acc[:, 128:384]
    kv_ref[:, 256:257] = lu_ref[...]
    kv_ref[:, 257:KVW] = jnp.zeros((acc.shape[0], KVW - 257), jnp.float32)
    sk_ref[...] = acc[:, 384:512]


def _tables(x, wcat, bcat, lu2d):
    return pl.pallas_call(
        _tables_body,
        grid=(N // BN,),
        in_specs=[
            pl.BlockSpec((BN, IN), lambda i: (i, 0)),
            pl.BlockSpec((IN, 512), lambda i: (0, 0)),
            pl.BlockSpec((1, 512), lambda i: (0, 0)),
            pl.BlockSpec((BN, 1), lambda i: (i, 0)),
        ],
        out_specs=[
            pl.BlockSpec((BN, 128), lambda i: (i, 0)),
            pl.BlockSpec((BN, KVW), lambda i: (i, 0)),
            pl.BlockSpec((BN, 128), lambda i: (i, 0)),
        ],
        out_shape=[
            jax.ShapeDtypeStruct((N, 128), jnp.float32),
            jax.ShapeDtypeStruct((N, KVW), jnp.float32),
            jax.ShapeDtypeStruct((N, 128), jnp.float32),
        ],
    )(x, wcat, bcat, lu2d)


def _gather_body(qt, kvt, src, dst,
                 qd_out, kvs_out,
                 sidx0, didx0, qbuf0, kvbuf0, semg0, semw0,
                 sidx1, didx1, qbuf1, kvbuf1, semg1, semw1,
                 sidxt, didxt, qbuft, kvbuft, semt):
    wid = lax.axis_index("s") * NC + lax.axis_index("c")
    base = wid * PER_TILE
    slots = ((sidx0, didx0, qbuf0, kvbuf0, semg0, semw0),
             (sidx1, didx1, qbuf1, kvbuf1, semg1, semw1))

    def issue(slot, i):
        sidx, didx, qbuf, kvbuf, semg, _ = slot
        off = base + i * CHUNK
        pltpu.sync_copy(src.at[pl.ds(off, CHUNK)], sidx)
        pltpu.sync_copy(dst.at[pl.ds(off, CHUNK)], didx)
        pltpu.async_copy(qt.at[didx], qbuf, semg)
        pltpu.async_copy(kvt.at[sidx], kvbuf, semg)

    def wait_g(slot):
        sidx, didx, qbuf, kvbuf, semg, _ = slot
        pltpu.make_async_copy(qt.at[didx], qbuf, semg).wait()
        pltpu.make_async_copy(kvt.at[sidx], kvbuf, semg).wait()

    def start_w(slot, i):
        _, _, qbuf, kvbuf, _, semw = slot
        off = base + i * CHUNK
        pltpu.async_copy(qbuf, qd_out.at[pl.ds(off, CHUNK)], semw)
        pltpu.async_copy(kvbuf, kvs_out.at[pl.ds(off, CHUNK)], semw)

    def wait_w(slot, i):
        _, _, qbuf, kvbuf, _, semw = slot
        off = base + i * CHUNK
        pltpu.make_async_copy(qbuf, qd_out.at[pl.ds(off, CHUNK)], semw).wait()
        pltpu.make_async_copy(kvbuf, kvs_out.at[pl.ds(off, CHUNK)], semw).wait()

    issue(slots[0], 0)

    def pair(g, carry):
        for b in (0, 1):
            k = g * 2 + b
            slot, oslot = slots[b], slots[1 - b]
            wait_g(slot)
            start_w(slot, k)

            @pl.when(k >= 1)
            def _():
                wait_w(oslot, k - 1)

            @pl.when(k + 1 < G_NCH)
            def _():
                issue(oslot, k + 1)

        return carry

    lax.fori_loop(0, G_NCH // 2, pair, 0)
    wait_w(slots[(G_NCH - 1) % 2], G_NCH - 1)

    off = base + G_NCH * CHUNK
    pltpu.sync_copy(src.at[pl.ds(off, G_TAIL)], sidxt)
    pltpu.sync_copy(dst.at[pl.ds(off, G_TAIL)], didxt)
    pltpu.async_copy(qt.at[didxt], qbuft, semt)
    pltpu.async_copy(kvt.at[sidxt], kvbuft, semt)
    pltpu.make_async_copy(qt.at[didxt], qbuft, semt).wait()
    pltpu.make_async_copy(kvt.at[sidxt], kvbuft, semt).wait()
    pltpu.sync_copy(qbuft, qd_out.at[pl.ds(off, G_TAIL)])
    pltpu.sync_copy(kvbuft, kvs_out.at[pl.ds(off, G_TAIL)])


def _gather(qt, kvt, src, dst):
    mesh = plsc.VectorSubcoreMesh(core_axis_name="c", subcore_axis_name="s")
    slot = [
        pltpu.VMEM((CHUNK,), jnp.int32),
        pltpu.VMEM((CHUNK,), jnp.int32),
        pltpu.VMEM((CHUNK, 128), jnp.float32),
        pltpu.VMEM((CHUNK, KVW), jnp.float32),
        pltpu.SemaphoreType.DMA,
        pltpu.SemaphoreType.DMA,
    ]
    tail = [
        pltpu.VMEM((G_TAIL,), jnp.int32),
        pltpu.VMEM((G_TAIL,), jnp.int32),
        pltpu.VMEM((G_TAIL, 128), jnp.float32),
        pltpu.VMEM((G_TAIL, KVW), jnp.float32),
        pltpu.SemaphoreType.DMA,
    ]
    fn = functools.partial(
        pl.kernel,
        out_type=(
            jax.ShapeDtypeStruct((E, 128), jnp.float32),
            jax.ShapeDtypeStruct((E, KVW), jnp.float32),
        ),
        mesh=mesh,
        scratch_types=slot + slot + tail,
    )(_gather_body)
    return fn(qt, kvt, src, dst)


def _edge_body(qd_ref, kvs_ref, t_ref, msg_ref,
               wtr_ref, btr_ref, wet_ref, wem_ref, shead_ref, bh_ref,
               pay0_ref, pay1_ref):
    kvs = kvs_ref[...]
    rel = kvs[:, 256:257] - t_ref[...]
    enc = jnp.cos(rel * wtr_ref[...] + btr_ref[...])
    e = jnp.dot(enc, wet_ref[...], preferred_element_type=jnp.float32)
    e = e + jnp.dot(msg_ref[...], wem_ref[...], preferred_element_type=jnp.float32)
    k = kvs[:, 0:128] + e
    v = kvs[:, 128:256] + e
    alpha = jnp.dot(qd_ref[...] * k, shead_ref[...],
                    preferred_element_type=jnp.float32) * 0.125
    ex = jnp.exp(alpha)
    exb = jnp.dot(ex, bh_ref[...], preferred_element_type=jnp.float32)
    pm = exb * v
    zer64 = jnp.zeros((pm.shape[0], 63), jnp.float32)
    pay0_ref[:, 0:64] = pm[:, 0:64]
    pay0_ref[:, 64:65] = ex[:, 0:1]
    pay0_ref[:, 65:128] = zer64
    pay1_ref[:, 0:64] = pm[:, 64:128]
    pay1_ref[:, 64:65] = ex[:, 1:2]
    pay1_ref[:, 65:128] = zer64


def _edges(qd, kvs, tf, msg, wtr, btr, wet, wem, shead, bh):
    return pl.pallas_call(
        _edge_body,
        grid=(E // BE,),
        in_specs=[
            pl.BlockSpec((BE, 128), lambda i: (i, 0)),
            pl.BlockSpec((BE, KVW), lambda i: (i, 0)),
            pl.BlockSpec((BE, 1), lambda i: (i, 0)),
            pl.BlockSpec((BE, MSG_DIM), lambda i: (i, 0)),
            pl.BlockSpec((1, 128), lambda i: (0, 0)),
            pl.BlockSpec((1, 128), lambda i: (0, 0)),
            pl.BlockSpec((128, 128), lambda i: (0, 0)),
            pl.BlockSpec((MSG_DIM, 128), lambda i: (0, 0)),
            pl.BlockSpec((128, HEADS), lambda i: (0, 0)),
            pl.BlockSpec((HEADS, 128), lambda i: (0, 0)),
        ],
        out_specs=[
            pl.BlockSpec((BE, 128), lambda i: (i, 0)),
            pl.BlockSpec((BE, 128), lambda i: (i, 0)),
        ],
        out_shape=[
            jax.ShapeDtypeStruct((E, 128), jnp.float32),
            jax.ShapeDtypeStruct((E, 128), jnp.float32),
        ],
    )(qd, kvs, tf, msg, wtr, btr, wet, wem, shead, bh)


def _scatter_body(pay0, pay1, dst, zer, out0, out1, table,
                  idx0, pbuf0, semp0, idx1, pbuf1, semp1, idxt, pbuft, sempt):
    cid = lax.axis_index("c")
    sid = lax.axis_index("s")
    base = sid * PER_TILE_1C
    slots = ((idx0, pbuf0, semp0), (idx1, pbuf1, semp1))

    @pl.when(sid == 0)
    def _init():
        pltpu.sync_copy(zer, table)

    plsc.subcore_barrier()

    def issue(slot, i):
        idxv, pbuf, semp = slot
        off = base + i * CHUNK
        pltpu.sync_copy(dst.at[pl.ds(off, CHUNK)], idxv)

        @pl.when(cid == 0)
        def _l0():
            pltpu.async_copy(pay0.at[pl.ds(off, CHUNK)], pbuf, semp)

        @pl.when(cid == 1)
        def _l1():
            pltpu.async_copy(pay1.at[pl.ds(off, CHUNK)], pbuf, semp)

    def wait_p(slot, i):
        idxv, pbuf, semp = slot
        off = base + i * CHUNK
        pltpu.make_async_copy(pay0.at[pl.ds(off, CHUNK)], pbuf, semp).wait()

    issue(slots[0], 0)

    def pair(g, carry):
        for b in (0, 1):
            k = g * 2 + b
            slot, oslot = slots[b], slots[1 - b]
            wait_p(slot, k)

            @pl.when(k + 1 < S_NCH)
            def _():
                issue(oslot, k + 1)

            idxv, pbuf, _ = slot
            pltpu.sync_copy(pbuf, table.at[idxv], add=True)
        return carry

    lax.fori_loop(0, S_NCH // 2, pair, 0)

    off = base + S_NCH * CHUNK
    pltpu.sync_copy(dst.at[pl.ds(off, S_TAIL)], idxt)

    @pl.when(cid == 0)
    def _t0():
        pltpu.sync_copy(pay0.at[pl.ds(off, S_TAIL)], pbuft)

    @pl.when(cid == 1)
    def _t1():
        pltpu.sync_copy(pay1.at[pl.ds(off, S_TAIL)], pbuft)

    pltpu.sync_copy(pbuft, table.at[idxt], add=True)

    plsc.subcore_barrier()

    @pl.when((sid == 0) & (cid == 0))
    def _dump0():
        pltpu.sync_copy(table, out0)

    @pl.when((sid == 0) & (cid == 1))
    def _dump1():
        pltpu.sync_copy(table, out1)


def _scatter(pay0, pay1, dst, zer):
    mesh = plsc.VectorSubcoreMesh(core_axis_name="c", subcore_axis_name="s")
    slot = [
        pltpu.VMEM((CHUNK,), jnp.int32),
        pltpu.VMEM((CHUNK, 128), jnp.float32),
        pltpu.SemaphoreType.DMA,
    ]
    tail = [
        pltpu.VMEM((S_TAIL,), jnp.int32),
        pltpu.VMEM((S_TAIL, 128), jnp.float32),
        pltpu.SemaphoreType.DMA,
    ]
    fn = functools.partial(
        pl.kernel,
        out_type=(
            jax.ShapeDtypeStruct((N, 128), jnp.float32),
            jax.ShapeDtypeStruct((N, 128), jnp.float32),
        ),
        mesh=mesh,
        scratch_types=[pltpu.VMEM_SHARED((N, 128), jnp.float32)] + slot + slot + tail,
    )(_scatter_body)
    return fn(pay0, pay1, dst, zer)


def _final_body(p0_ref, p1_ref, sk_ref, m0_ref, m1_ref, d0_ref, d1_ref, out_ref):
    p0 = p0_ref[...]
    p1 = p1_ref[...]
    main = jnp.dot(p0, m0_ref[...], preferred_element_type=jnp.float32)
    main = main + jnp.dot(p1, m1_ref[...], preferred_element_type=jnp.float32)
    den = jnp.dot(p0, d0_ref[...], preferred_element_type=jnp.float32)
    den = den + jnp.dot(p1, d1_ref[...], preferred_element_type=jnp.float32)
    agg = jnp.where(den > 0.0, main / den, 0.0)
    out_ref[...] = agg + sk_ref[...]


def _final(p0, p1, skip, m0, m1, d0, d1):
    return pl.pallas_call(
        _final_body,
        grid=(N // BN,),
        in_specs=[
            pl.BlockSpec((BN, 128), lambda i: (i, 0)),
            pl.BlockSpec((BN, 128), lambda i: (i, 0)),
            pl.BlockSpec((BN, 128), lambda i: (i, 0)),
            pl.BlockSpec((128, 128), lambda i: (0, 0)),
            pl.BlockSpec((128, 128), lambda i: (0, 0)),
            pl.BlockSpec((128, 128), lambda i: (0, 0)),
            pl.BlockSpec((128, 128), lambda i: (0, 0)),
        ],
        out_specs=pl.BlockSpec((BN, 128), lambda i: (i, 0)),
        out_shape=jax.ShapeDtypeStruct((N, 128), jnp.float32),
    )(p0, p1, skip, m0, m1, d0, d1)


def kernel(x, last_update, edge_index, t, msg, W_time, b_time,
           Wq, bq, Wk, bk, Wv, bv, We, Wskip, bskip):
    f32 = jnp.float32
    src = edge_index[0].astype(jnp.int32)
    dst = edge_index[1].astype(jnp.int32)
    lu2d = last_update.astype(f32)[:, None]
    tf = t.astype(f32)[:, None]

    wcat = jnp.concatenate([Wq, Wk, Wv, Wskip], axis=0).T.astype(f32)
    bcat = jnp.concatenate([bq, bk, bv, bskip])[None, :].astype(f32)

    wtr = jnp.zeros((1, 128), f32).at[0, :TIME_DIM].set(W_time[:, 0])
    btr = jnp.zeros((1, 128), f32).at[0, :TIME_DIM].set(b_time)
    wet = jnp.zeros((128, 128), f32).at[:TIME_DIM, :].set(We.T[:TIME_DIM, :])
    wem = We.T[TIME_DIM:, :].astype(f32)

    cols = jnp.arange(128) // C
    shead = (cols[:, None] == jnp.arange(HEADS)[None, :]).astype(f32)
    bh = shead.T

    i128 = jnp.arange(128)
    eye = jnp.eye(128, dtype=f32)
    m0 = jnp.where((i128[None, :] < 64) & (i128[:, None] == i128[None, :]), eye, 0.0)
    m1 = jnp.zeros((128, 128), f32).at[jnp.arange(64), jnp.arange(64) + 64].set(1.0)
    d0 = jnp.zeros((128, 128), f32).at[64, :].set((i128 < 64).astype(f32))
    d1 = jnp.zeros((128, 128), f32).at[64, :].set((i128 >= 64).astype(f32))

    zer = jnp.zeros((N, 128), f32)

    qt, kvt, skip = _tables(x, wcat, bcat, lu2d)
    qd, kvs = _gather(qt, kvt, src, dst)
    pay0, pay1 = _edges(qd, kvs, tf, msg, wtr, btr, wet, wem, shead, bh)
    p0, p1 = _scatter(pay0, pay1, dst, zer)
    return _final(p0, p1, skip, m0, m1, d0, d1)


# speedup vs baseline: 6.1241x; 1.0634x over previous
import functools

import jax
import jax.numpy as jnp
from jax import lax
from jax.experimental import pallas as pl
from jax.experimental.pallas import tpu as pltpu
from jax.experimental.pallas import tpu_sc as plsc

N = 10000
E = 320000
IN = 128
OUT = 128
HEADS = 2
C = OUT // HEADS
MSG_DIM = 16
TIME_DIM = 100
KVW = 256

NC = 2
NS = 16
NW = NC * NS
PER_TILE = E // NW
PER_TILE_1C = E // NS
CHUNK = 96
G_NCH = PER_TILE // CHUNK
G_TAIL = PER_TILE - G_NCH * CHUNK
S_NCH = PER_TILE_1C // CHUNK
S_TAIL = PER_TILE_1C - S_NCH * CHUNK

BE = 2000
BN = 2000


def _tables_body(x_ref, w_ref, b_ref, lu_ref, q_ref, kv_ref, sk_ref):
    acc = jnp.dot(x_ref[...], w_ref[...], preferred_element_type=jnp.float32)
    acc = acc + b_ref[...]
    q_ref[...] = acc[:, 0:128]
    hi_mask = jnp.uint32(0xFFFF0000)
    rne = jnp.uint32(0x7FFF)
    one = jnp.uint32(1)
    kb = jax.lax.bitcast_convert_type(acc[:, 128:256], jnp.uint32)
    vb = jax.lax.bitcast_convert_type(acc[:, 256:384], jnp.uint32)
    kr = (kb + rne + ((kb >> 16) & one)) & hi_mask
    vr = vb + rne + ((vb >> 16) & one)
    packed = kr | (vr >> 16)
    kv_ref[:, 0:128] = jax.lax.bitcast_convert_type(packed, jnp.float32)
    kv_ref[:, 128:129] = lu_ref[...]
    kv_ref[:, 129:KVW] = jnp.zeros((acc.shape[0], KVW - 129), jnp.float32)
    sk_ref[...] = acc[:, 384:512]


def _tables(x, wcat, bcat, lu2d):
    return pl.pallas_call(
        _tables_body,
        grid=(N // BN,),
        in_specs=[
            pl.BlockSpec((BN, IN), lambda i: (i, 0)),
            pl.BlockSpec((IN, 512), lambda i: (0, 0)),
            pl.BlockSpec((1, 512), lambda i: (0, 0)),
            pl.BlockSpec((BN, 1), lambda i: (i, 0)),
        ],
        out_specs=[
            pl.BlockSpec((BN, 128), lambda i: (i, 0)),
            pl.BlockSpec((BN, KVW), lambda i: (i, 0)),
            pl.BlockSpec((BN, 128), lambda i: (i, 0)),
        ],
        out_shape=[
            jax.ShapeDtypeStruct((N, 128), jnp.float32),
            jax.ShapeDtypeStruct((N, KVW), jnp.float32),
            jax.ShapeDtypeStruct((N, 128), jnp.float32),
        ],
    )(x, wcat, bcat, lu2d)


def _gather_body(qt, kvt, src, dst,
                 qd_out, kvs_out,
                 sidx0, didx0, qbuf0, kvbuf0, semg0, semw0,
                 sidx1, didx1, qbuf1, kvbuf1, semg1, semw1,
                 sidxt, didxt, qbuft, kvbuft, semt):
    wid = lax.axis_index("s") * NC + lax.axis_index("c")
    base = wid * PER_TILE
    slots = ((sidx0, didx0, qbuf0, kvbuf0, semg0, semw0),
             (sidx1, didx1, qbuf1, kvbuf1, semg1, semw1))

    def issue(slot, i):
        sidx, didx, qbuf, kvbuf, semg, _ = slot
        off = base + i * CHUNK
        pltpu.sync_copy(src.at[pl.ds(off, CHUNK)], sidx)
        pltpu.sync_copy(dst.at[pl.ds(off, CHUNK)], didx)
        pltpu.async_copy(qt.at[didx], qbuf, semg)
        pltpu.async_copy(kvt.at[sidx], kvbuf, semg)

    def wait_g(slot):
        sidx, didx, qbuf, kvbuf, semg, _ = slot
        pltpu.make_async_copy(qt.at[didx], qbuf, semg).wait()
        pltpu.make_async_copy(kvt.at[sidx], kvbuf, semg).wait()

    def start_w(slot, i):
        _, _, qbuf, kvbuf, _, semw = slot
        off = base + i * CHUNK
        pltpu.async_copy(qbuf, qd_out.at[pl.ds(off, CHUNK)], semw)
        pltpu.async_copy(kvbuf, kvs_out.at[pl.ds(off, CHUNK)], semw)

    def wait_w(slot, i):
        _, _, qbuf, kvbuf, _, semw = slot
        off = base + i * CHUNK
        pltpu.make_async_copy(qbuf, qd_out.at[pl.ds(off, CHUNK)], semw).wait()
        pltpu.make_async_copy(kvbuf, kvs_out.at[pl.ds(off, CHUNK)], semw).wait()

    issue(slots[0], 0)

    def pair(g, carry):
        for b in (0, 1):
            k = g * 2 + b
            slot, oslot = slots[b], slots[1 - b]
            wait_g(slot)
            start_w(slot, k)

            @pl.when(k >= 1)
            def _():
                wait_w(oslot, k - 1)

            @pl.when(k + 1 < G_NCH)
            def _():
                issue(oslot, k + 1)

        return carry

    lax.fori_loop(0, G_NCH // 2, pair, 0)
    wait_w(slots[(G_NCH - 1) % 2], G_NCH - 1)

    off = base + G_NCH * CHUNK
    pltpu.sync_copy(src.at[pl.ds(off, G_TAIL)], sidxt)
    pltpu.sync_copy(dst.at[pl.ds(off, G_TAIL)], didxt)
    pltpu.async_copy(qt.at[didxt], qbuft, semt)
    pltpu.async_copy(kvt.at[sidxt], kvbuft, semt)
    pltpu.make_async_copy(qt.at[didxt], qbuft, semt).wait()
    pltpu.make_async_copy(kvt.at[sidxt], kvbuft, semt).wait()
    pltpu.sync_copy(qbuft, qd_out.at[pl.ds(off, G_TAIL)])
    pltpu.sync_copy(kvbuft, kvs_out.at[pl.ds(off, G_TAIL)])


def _gather(qt, kvt, src, dst):
    mesh = plsc.VectorSubcoreMesh(core_axis_name="c", subcore_axis_name="s")
    slot = [
        pltpu.VMEM((CHUNK,), jnp.int32),
        pltpu.VMEM((CHUNK,), jnp.int32),
        pltpu.VMEM((CHUNK, 128), jnp.float32),
        pltpu.VMEM((CHUNK, KVW), jnp.float32),
        pltpu.SemaphoreType.DMA,
        pltpu.SemaphoreType.DMA,
    ]
    tail = [
        pltpu.VMEM((G_TAIL,), jnp.int32),
        pltpu.VMEM((G_TAIL,), jnp.int32),
        pltpu.VMEM((G_TAIL, 128), jnp.float32),
        pltpu.VMEM((G_TAIL, KVW), jnp.float32),
        pltpu.SemaphoreType.DMA,
    ]
    fn = functools.partial(
        pl.kernel,
        out_type=(
            jax.ShapeDtypeStruct((E, 128), jnp.float32),
            jax.ShapeDtypeStruct((E, KVW), jnp.float32),
        ),
        mesh=mesh,
        scratch_types=slot + slot + tail,
    )(_gather_body)
    return fn(qt, kvt, src, dst)


def _edge_body(qd_ref, kvs_ref, t_ref, msg_ref,
               wtr_ref, btr_ref, wet_ref, wem_ref, shead_ref, bh_ref,
               pay0_ref, pay1_ref):
    kvs = kvs_ref[...]
    pk = jax.lax.bitcast_convert_type(kvs[:, 0:128], jnp.uint32)
    k = jax.lax.bitcast_convert_type(pk & jnp.uint32(0xFFFF0000), jnp.float32)
    v = jax.lax.bitcast_convert_type(pk << 16, jnp.float32)
    rel = kvs[:, 128:129] - t_ref[...]
    enc = jnp.cos(rel * wtr_ref[...] + btr_ref[...])
    e = jnp.dot(enc, wet_ref[...], preferred_element_type=jnp.float32)
    e = e + jnp.dot(msg_ref[...], wem_ref[...], preferred_element_type=jnp.float32)
    k = k + e
    v = v + e
    alpha = jnp.dot(qd_ref[...] * k, shead_ref[...],
                    preferred_element_type=jnp.float32) * 0.125
    ex = jnp.exp(alpha)
    exb = jnp.dot(ex, bh_ref[...], preferred_element_type=jnp.float32)
    pm = exb * v
    zer64 = jnp.zeros((pm.shape[0], 63), jnp.float32)
    pay0_ref[:, 0:64] = pm[:, 0:64]
    pay0_ref[:, 64:65] = ex[:, 0:1]
    pay0_ref[:, 65:128] = zer64
    pay1_ref[:, 0:64] = pm[:, 64:128]
    pay1_ref[:, 64:65] = ex[:, 1:2]
    pay1_ref[:, 65:128] = zer64


def _edges(qd, kvs, tf, msg, wtr, btr, wet, wem, shead, bh):
    return pl.pallas_call(
        _edge_body,
        grid=(E // BE,),
        in_specs=[
            pl.BlockSpec((BE, 128), lambda i: (i, 0)),
            pl.BlockSpec((BE, KVW), lambda i: (i, 0)),
            pl.BlockSpec((BE, 1), lambda i: (i, 0)),
            pl.BlockSpec((BE, MSG_DIM), lambda i: (i, 0)),
            pl.BlockSpec((1, 128), lambda i: (0, 0)),
            pl.BlockSpec((1, 128), lambda i: (0, 0)),
            pl.BlockSpec((128, 128), lambda i: (0, 0)),
            pl.BlockSpec((MSG_DIM, 128), lambda i: (0, 0)),
            pl.BlockSpec((128, HEADS), lambda i: (0, 0)),
            pl.BlockSpec((HEADS, 128), lambda i: (0, 0)),
        ],
        out_specs=[
            pl.BlockSpec((BE, 128), lambda i: (i, 0)),
            pl.BlockSpec((BE, 128), lambda i: (i, 0)),
        ],
        out_shape=[
            jax.ShapeDtypeStruct((E, 128), jnp.float32),
            jax.ShapeDtypeStruct((E, 128), jnp.float32),
        ],
    )(qd, kvs, tf, msg, wtr, btr, wet, wem, shead, bh)


def _scatter_body(pay0, pay1, dst, zer, out0, out1, table,
                  idx0, pbuf0, semp0, idx1, pbuf1, semp1, idxt, pbuft, sempt):
    cid = lax.axis_index("c")
    sid = lax.axis_index("s")
    base = sid * PER_TILE_1C
    slots = ((idx0, pbuf0, semp0), (idx1, pbuf1, semp1))

    @pl.when(sid == 0)
    def _init():
        pltpu.sync_copy(zer, table)

    plsc.subcore_barrier()

    def issue(slot, i):
        idxv, pbuf, semp = slot
        off = base + i * CHUNK
        pltpu.sync_copy(dst.at[pl.ds(off, CHUNK)], idxv)

        @pl.when(cid == 0)
        def _l0():
            pltpu.async_copy(pay0.at[pl.ds(off, CHUNK)], pbuf, semp)

        @pl.when(cid == 1)
        def _l1():
            pltpu.async_copy(pay1.at[pl.ds(off, CHUNK)], pbuf, semp)

    def wait_p(slot, i):
        idxv, pbuf, semp = slot
        off = base + i * CHUNK
        pltpu.make_async_copy(pay0.at[pl.ds(off, CHUNK)], pbuf, semp).wait()

    issue(slots[0], 0)

    def pair(g, carry):
        for b in (0, 1):
            k = g * 2 + b
            slot, oslot = slots[b], slots[1 - b]
            wait_p(slot, k)

            @pl.when(k + 1 < S_NCH)
            def _():
                issue(oslot, k + 1)

            idxv, pbuf, _ = slot
            pltpu.sync_copy(pbuf, table.at[idxv], add=True)
        return carry

    lax.fori_loop(0, S_NCH // 2, pair, 0)

    off = base + S_NCH * CHUNK
    pltpu.sync_copy(dst.at[pl.ds(off, S_TAIL)], idxt)

    @pl.when(cid == 0)
    def _t0():
        pltpu.sync_copy(pay0.at[pl.ds(off, S_TAIL)], pbuft)

    @pl.when(cid == 1)
    def _t1():
        pltpu.sync_copy(pay1.at[pl.ds(off, S_TAIL)], pbuft)

    pltpu.sync_copy(pbuft, table.at[idxt], add=True)

    plsc.subcore_barrier()

    @pl.when((sid == 0) & (cid == 0))
    def _dump0():
        pltpu.sync_copy(table, out0)

    @pl.when((sid == 0) & (cid == 1))
    def _dump1():
        pltpu.sync_copy(table, out1)


def _scatter(pay0, pay1, dst, zer):
    mesh = plsc.VectorSubcoreMesh(core_axis_name="c", subcore_axis_name="s")
    slot = [
        pltpu.VMEM((CHUNK,), jnp.int32),
        pltpu.VMEM((CHUNK, 128), jnp.float32),
        pltpu.SemaphoreType.DMA,
    ]
    tail = [
        pltpu.VMEM((S_TAIL,), jnp.int32),
        pltpu.VMEM((S_TAIL, 128), jnp.float32),
        pltpu.SemaphoreType.DMA,
    ]
    fn = functools.partial(
        pl.kernel,
        out_type=(
            jax.ShapeDtypeStruct((N, 128), jnp.float32),
            jax.ShapeDtypeStruct((N, 128), jnp.float32),
        ),
        mesh=mesh,
        scratch_types=[pltpu.VMEM_SHARED((N, 128), jnp.float32)] + slot + slot + tail,
    )(_scatter_body)
    return fn(pay0, pay1, dst, zer)


def _final_body(p0_ref, p1_ref, sk_ref, m0_ref, m1_ref, d0_ref, d1_ref, out_ref):
    p0 = p0_ref[...]
    p1 = p1_ref[...]
    main = jnp.dot(p0, m0_ref[...], preferred_element_type=jnp.float32)
    main = main + jnp.dot(p1, m1_ref[...], preferred_element_type=jnp.float32)
    den = jnp.dot(p0, d0_ref[...], preferred_element_type=jnp.float32)
    den = den + jnp.dot(p1, d1_ref[...], preferred_element_type=jnp.float32)
    agg = jnp.where(den > 0.0, main / den, 0.0)
    out_ref[...] = agg + sk_ref[...]


def _final(p0, p1, skip, m0, m1, d0, d1):
    return pl.pallas_call(
        _final_body,
        grid=(N // BN,),
        in_specs=[
            pl.BlockSpec((BN, 128), lambda i: (i, 0)),
            pl.BlockSpec((BN, 128), lambda i: (i, 0)),
            pl.BlockSpec((BN, 128), lambda i: (i, 0)),
            pl.BlockSpec((128, 128), lambda i: (0, 0)),
            pl.BlockSpec((128, 128), lambda i: (0, 0)),
            pl.BlockSpec((128, 128), lambda i: (0, 0)),
            pl.BlockSpec((128, 128), lambda i: (0, 0)),
        ],
        out_specs=pl.BlockSpec((BN, 128), lambda i: (i, 0)),
        out_shape=jax.ShapeDtypeStruct((N, 128), jnp.float32),
    )(p0, p1, skip, m0, m1, d0, d1)


def kernel(x, last_update, edge_index, t, msg, W_time, b_time,
           Wq, bq, Wk, bk, Wv, bv, We, Wskip, bskip):
    f32 = jnp.float32
    src = edge_index[0].astype(jnp.int32)
    dst = edge_index[1].astype(jnp.int32)
    lu2d = last_update.astype(f32)[:, None]
    tf = t.astype(f32)[:, None]

    wcat = jnp.concatenate([Wq, Wk, Wv, Wskip], axis=0).T.astype(f32)
    bcat = jnp.concatenate([bq, bk, bv, bskip])[None, :].astype(f32)

    wtr = jnp.zeros((1, 128), f32).at[0, :TIME_DIM].set(W_time[:, 0])
    btr = jnp.zeros((1, 128), f32).at[0, :TIME_DIM].set(b_time)
    wet = jnp.zeros((128, 128), f32).at[:TIME_DIM, :].set(We.T[:TIME_DIM, :])
    wem = We.T[TIME_DIM:, :].astype(f32)

    cols = jnp.arange(128) // C
    shead = (cols[:, None] == jnp.arange(HEADS)[None, :]).astype(f32)
    bh = shead.T

    i128 = jnp.arange(128)
    eye = jnp.eye(128, dtype=f32)
    m0 = jnp.where((i128[None, :] < 64) & (i128[:, None] == i128[None, :]), eye, 0.0)
    m1 = jnp.zeros((128, 128), f32).at[jnp.arange(64), jnp.arange(64) + 64].set(1.0)
    d0 = jnp.zeros((128, 128), f32).at[64, :].set((i128 < 64).astype(f32))
    d1 = jnp.zeros((128, 128), f32).at[64, :].set((i128 >= 64).astype(f32))

    zer = jnp.zeros((N, 128), f32)

    qt, kvt, skip = _tables(x, wcat, bcat, lu2d)
    qd, kvs = _gather(qt, kvt, src, dst)
    pay0, pay1 = _edges(qd, kvs, tf, msg, wtr, btr, wet, wem, shead, bh)
    p0, p1 = _scatter(pay0, pay1, dst, zer)
    return _final(p0, p1, skip, m0, m1, d0, d1)


# speedup vs baseline: 6.3508x; 1.0370x over previous
import functools

import jax
import jax.numpy as jnp
from jax import lax
from jax.experimental import pallas as pl
from jax.experimental.pallas import tpu as pltpu
from jax.experimental.pallas import tpu_sc as plsc

N = 10000
E = 320000
IN = 128
OUT = 128
HEADS = 2
C = OUT // HEADS
MSG_DIM = 16
TIME_DIM = 100
KVW = 256

NC = 2
NS = 16
NW = NC * NS
PER_TILE = E // NW
PER_TILE_1C = E // NS
CHUNK = 128
G_NCH = PER_TILE // CHUNK
G_TAIL = PER_TILE - G_NCH * CHUNK
S_NCH = PER_TILE_1C // CHUNK
S_TAIL = PER_TILE_1C - S_NCH * CHUNK

BE = 2000
BN = 2000


def _tables_body(x_ref, w_ref, b_ref, lu_ref, q_ref, kv_ref, sk_ref):
    acc = jnp.dot(x_ref[...], w_ref[...], preferred_element_type=jnp.float32)
    acc = acc + b_ref[...]
    q_ref[...] = acc[:, 0:128]
    hi_mask = jnp.uint32(0xFFFF0000)
    rne = jnp.uint32(0x7FFF)
    one = jnp.uint32(1)
    kb = jax.lax.bitcast_convert_type(acc[:, 128:256], jnp.uint32)
    vb = jax.lax.bitcast_convert_type(acc[:, 256:384], jnp.uint32)
    kr = (kb + rne + ((kb >> 16) & one)) & hi_mask
    vr = vb + rne + ((vb >> 16) & one)
    packed = kr | (vr >> 16)
    kv_ref[:, 0:128] = jax.lax.bitcast_convert_type(packed, jnp.float32)
    kv_ref[:, 128:129] = lu_ref[...]
    kv_ref[:, 129:KVW] = jnp.zeros((acc.shape[0], KVW - 129), jnp.float32)
    sk_ref[...] = acc[:, 384:512]


def _tables(x, wcat, bcat, lu2d):
    return pl.pallas_call(
        _tables_body,
        grid=(N // BN,),
        in_specs=[
            pl.BlockSpec((BN, IN), lambda i: (i, 0)),
            pl.BlockSpec((IN, 512), lambda i: (0, 0)),
            pl.BlockSpec((1, 512), lambda i: (0, 0)),
            pl.BlockSpec((BN, 1), lambda i: (i, 0)),
        ],
        out_specs=[
            pl.BlockSpec((BN, 128), lambda i: (i, 0)),
            pl.BlockSpec((BN, KVW), lambda i: (i, 0)),
            pl.BlockSpec((BN, 128), lambda i: (i, 0)),
        ],
        out_shape=[
            jax.ShapeDtypeStruct((N, 128), jnp.float32),
            jax.ShapeDtypeStruct((N, KVW), jnp.float32),
            jax.ShapeDtypeStruct((N, 128), jnp.float32),
        ],
    )(x, wcat, bcat, lu2d)


def _gather_body(qt, kvt, src, dst,
                 qd_out, kvs_out,
                 sidx0, didx0, qbuf0, kvbuf0, semg0, semw0,
                 sidx1, didx1, qbuf1, kvbuf1, semg1, semw1,
                 sidxt, didxt, qbuft, kvbuft, semt):
    wid = lax.axis_index("s") * NC + lax.axis_index("c")
    base = wid * PER_TILE
    slots = ((sidx0, didx0, qbuf0, kvbuf0, semg0, semw0),
             (sidx1, didx1, qbuf1, kvbuf1, semg1, semw1))

    def issue(slot, i):
        sidx, didx, qbuf, kvbuf, semg, _ = slot
        off = base + i * CHUNK
        pltpu.sync_copy(src.at[pl.ds(off, CHUNK)], sidx)
        pltpu.sync_copy(dst.at[pl.ds(off, CHUNK)], didx)
        pltpu.async_copy(qt.at[didx], qbuf, semg)
        pltpu.async_copy(kvt.at[sidx], kvbuf, semg)

    def wait_g(slot):
        sidx, didx, qbuf, kvbuf, semg, _ = slot
        pltpu.make_async_copy(qt.at[didx], qbuf, semg).wait()
        pltpu.make_async_copy(kvt.at[sidx], kvbuf, semg).wait()

    def start_w(slot, i):
        _, _, qbuf, kvbuf, _, semw = slot
        off = base + i * CHUNK
        pltpu.async_copy(qbuf, qd_out.at[pl.ds(off, CHUNK)], semw)
        pltpu.async_copy(kvbuf, kvs_out.at[pl.ds(off, CHUNK)], semw)

    def wait_w(slot, i):
        _, _, qbuf, kvbuf, _, semw = slot
        off = base + i * CHUNK
        pltpu.make_async_copy(qbuf, qd_out.at[pl.ds(off, CHUNK)], semw).wait()
        pltpu.make_async_copy(kvbuf, kvs_out.at[pl.ds(off, CHUNK)], semw).wait()

    issue(slots[0], 0)

    def pair(g, carry):
        for b in (0, 1):
            k = g * 2 + b
            slot, oslot = slots[b], slots[1 - b]
            wait_g(slot)
            start_w(slot, k)

            @pl.when(k >= 1)
            def _():
                wait_w(oslot, k - 1)

            @pl.when(k + 1 < G_NCH)
            def _():
                issue(oslot, k + 1)

        return carry

    lax.fori_loop(0, G_NCH // 2, pair, 0)
    wait_w(slots[(G_NCH - 1) % 2], G_NCH - 1)

    off = base + G_NCH * CHUNK
    pltpu.sync_copy(src.at[pl.ds(off, G_TAIL)], sidxt)
    pltpu.sync_copy(dst.at[pl.ds(off, G_TAIL)], didxt)
    pltpu.async_copy(qt.at[didxt], qbuft, semt)
    pltpu.async_copy(kvt.at[sidxt], kvbuft, semt)
    pltpu.make_async_copy(qt.at[didxt], qbuft, semt).wait()
    pltpu.make_async_copy(kvt.at[sidxt], kvbuft, semt).wait()
    pltpu.sync_copy(qbuft, qd_out.at[pl.ds(off, G_TAIL)])
    pltpu.sync_copy(kvbuft, kvs_out.at[pl.ds(off, G_TAIL)])


def _gather(qt, kvt, src, dst):
    mesh = plsc.VectorSubcoreMesh(core_axis_name="c", subcore_axis_name="s")
    slot = [
        pltpu.VMEM((CHUNK,), jnp.int32),
        pltpu.VMEM((CHUNK,), jnp.int32),
        pltpu.VMEM((CHUNK, 128), jnp.float32),
        pltpu.VMEM((CHUNK, KVW), jnp.float32),
        pltpu.SemaphoreType.DMA,
        pltpu.SemaphoreType.DMA,
    ]
    tail = [
        pltpu.VMEM((G_TAIL,), jnp.int32),
        pltpu.VMEM((G_TAIL,), jnp.int32),
        pltpu.VMEM((G_TAIL, 128), jnp.float32),
        pltpu.VMEM((G_TAIL, KVW), jnp.float32),
        pltpu.SemaphoreType.DMA,
    ]
    fn = functools.partial(
        pl.kernel,
        out_type=(
            jax.ShapeDtypeStruct((E, 128), jnp.float32),
            jax.ShapeDtypeStruct((E, KVW), jnp.float32),
        ),
        mesh=mesh,
        scratch_types=slot + slot + tail,
    )(_gather_body)
    return fn(qt, kvt, src, dst)


def _edge_body(qd_ref, kvs_ref, t_ref, msg_ref,
               wtr_ref, btr_ref, wet_ref, wem_ref, shead_ref, bh_ref,
               pay0_ref, pay1_ref):
    kvs = kvs_ref[...]
    pk = jax.lax.bitcast_convert_type(kvs[:, 0:128], jnp.uint32)
    k = jax.lax.bitcast_convert_type(pk & jnp.uint32(0xFFFF0000), jnp.float32)
    v = jax.lax.bitcast_convert_type(pk << 16, jnp.float32)
    rel = kvs[:, 128:129] - t_ref[...]
    enc = jnp.cos(rel * wtr_ref[...] + btr_ref[...])
    e = jnp.dot(enc, wet_ref[...], preferred_element_type=jnp.float32)
    e = e + jnp.dot(msg_ref[...], wem_ref[...], preferred_element_type=jnp.float32)
    k = k + e
    v = v + e
    alpha = jnp.dot(qd_ref[...] * k, shead_ref[...],
                    preferred_element_type=jnp.float32) * 0.125
    ex = jnp.exp(alpha)
    exb = jnp.dot(ex, bh_ref[...], preferred_element_type=jnp.float32)
    pm = exb * v
    zer64 = jnp.zeros((pm.shape[0], 63), jnp.float32)
    pay0_ref[:, 0:64] = pm[:, 0:64]
    pay0_ref[:, 64:65] = ex[:, 0:1]
    pay0_ref[:, 65:128] = zer64
    pay1_ref[:, 0:64] = pm[:, 64:128]
    pay1_ref[:, 64:65] = ex[:, 1:2]
    pay1_ref[:, 65:128] = zer64


def _edges(qd, kvs, tf, msg, wtr, btr, wet, wem, shead, bh):
    return pl.pallas_call(
        _edge_body,
        grid=(E // BE,),
        in_specs=[
            pl.BlockSpec((BE, 128), lambda i: (i, 0)),
            pl.BlockSpec((BE, KVW), lambda i: (i, 0)),
            pl.BlockSpec((BE, 1), lambda i: (i, 0)),
            pl.BlockSpec((BE, MSG_DIM), lambda i: (i, 0)),
            pl.BlockSpec((1, 128), lambda i: (0, 0)),
            pl.BlockSpec((1, 128), lambda i: (0, 0)),
            pl.BlockSpec((128, 128), lambda i: (0, 0)),
            pl.BlockSpec((MSG_DIM, 128), lambda i: (0, 0)),
            pl.BlockSpec((128, HEADS), lambda i: (0, 0)),
            pl.BlockSpec((HEADS, 128), lambda i: (0, 0)),
        ],
        out_specs=[
            pl.BlockSpec((BE, 128), lambda i: (i, 0)),
            pl.BlockSpec((BE, 128), lambda i: (i, 0)),
        ],
        out_shape=[
            jax.ShapeDtypeStruct((E, 128), jnp.float32),
            jax.ShapeDtypeStruct((E, 128), jnp.float32),
        ],
    )(qd, kvs, tf, msg, wtr, btr, wet, wem, shead, bh)


def _scatter_body(pay0, pay1, dst, zer, out0, out1, table,
                  idx0, pbuf0, semp0, idx1, pbuf1, semp1, idxt, pbuft, sempt):
    cid = lax.axis_index("c")
    sid = lax.axis_index("s")
    base = sid * PER_TILE_1C
    slots = ((idx0, pbuf0, semp0), (idx1, pbuf1, semp1))

    @pl.when(sid == 0)
    def _init():
        pltpu.sync_copy(zer, table)

    plsc.subcore_barrier()

    def issue(slot, i):
        idxv, pbuf, semp = slot
        off = base + i * CHUNK
        pltpu.sync_copy(dst.at[pl.ds(off, CHUNK)], idxv)

        @pl.when(cid == 0)
        def _l0():
            pltpu.async_copy(pay0.at[pl.ds(off, CHUNK)], pbuf, semp)

        @pl.when(cid == 1)
        def _l1():
            pltpu.async_copy(pay1.at[pl.ds(off, CHUNK)], pbuf, semp)

    def wait_p(slot, i):
        idxv, pbuf, semp = slot
        off = base + i * CHUNK
        pltpu.make_async_copy(pay0.at[pl.ds(off, CHUNK)], pbuf, semp).wait()

    issue(slots[0], 0)

    def pair(g, carry):
        for b in (0, 1):
            k = g * 2 + b
            slot, oslot = slots[b], slots[1 - b]
            wait_p(slot, k)

            @pl.when(k + 1 < S_NCH)
            def _():
                issue(oslot, k + 1)

            idxv, pbuf, _ = slot
            pltpu.sync_copy(pbuf, table.at[idxv], add=True)
        return carry

    lax.fori_loop(0, S_NCH // 2, pair, 0)

    off = base + S_NCH * CHUNK
    pltpu.sync_copy(dst.at[pl.ds(off, S_TAIL)], idxt)

    @pl.when(cid == 0)
    def _t0():
        pltpu.sync_copy(pay0.at[pl.ds(off, S_TAIL)], pbuft)

    @pl.when(cid == 1)
    def _t1():
        pltpu.sync_copy(pay1.at[pl.ds(off, S_TAIL)], pbuft)

    pltpu.sync_copy(pbuft, table.at[idxt], add=True)

    plsc.subcore_barrier()

    @pl.when((sid == 0) & (cid == 0))
    def _dump0():
        pltpu.sync_copy(table, out0)

    @pl.when((sid == 0) & (cid == 1))
    def _dump1():
        pltpu.sync_copy(table, out1)


def _scatter(pay0, pay1, dst, zer):
    mesh = plsc.VectorSubcoreMesh(core_axis_name="c", subcore_axis_name="s")
    slot = [
        pltpu.VMEM((CHUNK,), jnp.int32),
        pltpu.VMEM((CHUNK, 128), jnp.float32),
        pltpu.SemaphoreType.DMA,
    ]
    tail = [
        pltpu.VMEM((S_TAIL,), jnp.int32),
        pltpu.VMEM((S_TAIL, 128), jnp.float32),
        pltpu.SemaphoreType.DMA,
    ]
    fn = functools.partial(
        pl.kernel,
        out_type=(
            jax.ShapeDtypeStruct((N, 128), jnp.float32),
            jax.ShapeDtypeStruct((N, 128), jnp.float32),
        ),
        mesh=mesh,
        scratch_types=[pltpu.VMEM_SHARED((N, 128), jnp.float32)] + slot + slot + tail,
    )(_scatter_body)
    return fn(pay0, pay1, dst, zer)


def _final_body(p0_ref, p1_ref, sk_ref, m0_ref, m1_ref, d0_ref, d1_ref, out_ref):
    p0 = p0_ref[...]
    p1 = p1_ref[...]
    main = jnp.dot(p0, m0_ref[...], preferred_element_type=jnp.float32)
    main = main + jnp.dot(p1, m1_ref[...], preferred_element_type=jnp.float32)
    den = jnp.dot(p0, d0_ref[...], preferred_element_type=jnp.float32)
    den = den + jnp.dot(p1, d1_ref[...], preferred_element_type=jnp.float32)
    agg = jnp.where(den > 0.0, main / den, 0.0)
    out_ref[...] = agg + sk_ref[...]


def _final(p0, p1, skip, m0, m1, d0, d1):
    return pl.pallas_call(
        _final_body,
        grid=(N // BN,),
        in_specs=[
            pl.BlockSpec((BN, 128), lambda i: (i, 0)),
            pl.BlockSpec((BN, 128), lambda i: (i, 0)),
            pl.BlockSpec((BN, 128), lambda i: (i, 0)),
            pl.BlockSpec((128, 128), lambda i: (0, 0)),
            pl.BlockSpec((128, 128), lambda i: (0, 0)),
            pl.BlockSpec((128, 128), lambda i: (0, 0)),
            pl.BlockSpec((128, 128), lambda i: (0, 0)),
        ],
        out_specs=pl.BlockSpec((BN, 128), lambda i: (i, 0)),
        out_shape=jax.ShapeDtypeStruct((N, 128), jnp.float32),
    )(p0, p1, skip, m0, m1, d0, d1)


def kernel(x, last_update, edge_index, t, msg, W_time, b_time,
           Wq, bq, Wk, bk, Wv, bv, We, Wskip, bskip):
    f32 = jnp.float32
    src = edge_index[0].astype(jnp.int32)
    dst = edge_index[1].astype(jnp.int32)
    lu2d = last_update.astype(f32)[:, None]
    tf = t.astype(f32)[:, None]

    wcat = jnp.concatenate([Wq, Wk, Wv, Wskip], axis=0).T.astype(f32)
    bcat = jnp.concatenate([bq, bk, bv, bskip])[None, :].astype(f32)

    wtr = jnp.zeros((1, 128), f32).at[0, :TIME_DIM].set(W_time[:, 0])
    btr = jnp.zeros((1, 128), f32).at[0, :TIME_DIM].set(b_time)
    wet = jnp.zeros((128, 128), f32).at[:TIME_DIM, :].set(We.T[:TIME_DIM, :])
    wem = We.T[TIME_DIM:, :].astype(f32)

    cols = jnp.arange(128) // C
    shead = (cols[:, None] == jnp.arange(HEADS)[None, :]).astype(f32)
    bh = shead.T

    i128 = jnp.arange(128)
    eye = jnp.eye(128, dtype=f32)
    m0 = jnp.where((i128[None, :] < 64) & (i128[:, None] == i128[None, :]), eye, 0.0)
    m1 = jnp.zeros((128, 128), f32).at[jnp.arange(64), jnp.arange(64) + 64].set(1.0)
    d0 = jnp.zeros((128, 128), f32).at[64, :].set((i128 < 64).astype(f32))
    d1 = jnp.zeros((128, 128), f32).at[64, :].set((i128 >= 64).astype(f32))

    zer = jnp.zeros((N, 128), f32)

    qt, kvt, skip = _tables(x, wcat, bcat, lu2d)
    qd, kvs = _gather(qt, kvt, src, dst)
    pay0, pay1 = _edges(qd, kvs, tf, msg, wtr, btr, wet, wem, shead, bh)
    p0, p1 = _scatter(pay0, pay1, dst, zer)
    return _final(p0, p1, skip, m0, m1, d0, d1)
